# baseline (speedup 1.0000x reference)
import functools

import jax
import jax.numpy as jnp
from jax import lax
from jax.experimental import pallas as pl
from jax.experimental.pallas import tpu as pltpu
from jax.experimental.pallas import tpu_sc as plsc

N_NODES = 8
B = 64
K = 2
P = 512
S = 2
R = 102
NOISE_DIM = 16
H = 64
KP = K * P
Q = B * K * R
OPB = K * R * S
NPAIR = 16


def _tc_body(noise_ref, bw_ref, tw1_ref, tb1_ref, tw2_ref, tb2_ref,
             delta_ref, cum_ref):
    nz = noise_ref[0]
    w1 = tw1_ref[0]
    h = jnp.dot(nz, w1, preferred_element_type=jnp.float32) + tb1_ref[0]
    h = jnp.maximum(h, 0.0)
    d = jnp.dot(h, tw2_ref[0], preferred_element_type=jnp.float32) + tb2_ref[0]
    delta_ref[0] = d

    w = bw_ref[0]
    t = jnp.sum(w, axis=1, keepdims=True)
    c = w / t
    lane = lax.broadcasted_iota(jnp.int32, (B, KP), 1)
    s = 1
    while s < KP:
        c = c + jnp.where(lane >= s, pltpu.roll(c, s, 1), 0.0)
        s *= 2
    cum_ref[0] = c


def _tc_call(noise3, bw3, tw1, tb1r, tw2, tb2r):
    return pl.pallas_call(
        _tc_body,
        grid=(N_NODES,),
        in_specs=[
            pl.BlockSpec((1, Q, NOISE_DIM), lambda i: (i, 0, 0)),
            pl.BlockSpec((1, B, KP), lambda i: (i, 0, 0)),
            pl.BlockSpec((1, NOISE_DIM, H), lambda i: (i, 0, 0)),
            pl.BlockSpec((1, 1, H), lambda i: (i, 0, 0)),
            pl.BlockSpec((1, H, S), lambda i: (i, 0, 0)),
            pl.BlockSpec((1, 1, S), lambda i: (i, 0, 0)),
        ],
        out_specs=[
            pl.BlockSpec((1, Q, S), lambda i: (i, 0, 0)),
            pl.BlockSpec((1, B, KP), lambda i: (i, 0, 0)),
        ],
        out_shape=[
            jax.ShapeDtypeStruct((N_NODES, Q, S), jnp.float32),
            jax.ShapeDtypeStruct((N_NODES, B, KP), jnp.float32),
        ],
    )(noise3, bw3, tw1, tb1r, tw2, tb2r)


def _final_body(m_ref, g_ref, d_ref, o_ref):
    o_ref[0, :, :R * S] = jnp.clip(g_ref[0] + d_ref[0], -1.0, 1.0)
    o_ref[0, :, R * S:] = m_ref[0, :, R * S:]


def _final_call(msg3, g3, d3):
    return pl.pallas_call(
        _final_body,
        grid=(N_NODES,),
        in_specs=[
            pl.BlockSpec((1, B * K, P * S), lambda i: (i, 0, 0)),
            pl.BlockSpec((1, B * K, R * S), lambda i: (i, 0, 0)),
            pl.BlockSpec((1, B * K, R * S), lambda i: (i, 0, 0)),
        ],
        out_specs=pl.BlockSpec((1, B * K, P * S), lambda i: (i, 0, 0)),
        out_shape=jax.ShapeDtypeStruct((N_NODES, B * K, P * S), jnp.float32),
    )(msg3, g3, d3)


def _issue2(cum_hbm, bp_hbm, cum_v, bp_v, sem, p):
    pltpu.async_copy(cum_hbm.at[pl.ds(p * KP, KP)], cum_v, sem)
    pltpu.async_copy(bp_hbm.at[pl.ds(p * KP * S, KP * S)], bp_v, sem)


def _drain2(cum_hbm, bp_hbm, cum_v, bp_v, sem):
    pltpu.make_async_copy(cum_hbm.at[pl.ds(0, KP)], cum_v, sem).wait()
    pltpu.make_async_copy(bp_hbm.at[pl.ds(0, KP * S)], bp_v, sem).wait()


def _sc_body(cum_hbm, bp_hbm, u_hbm, out_hbm,
             cum_v0, bp_v0, cum_v1, bp_v1,
             out_v0, out_v1, u_v,
             sem0, sem1, osem0, osem1):
    nc = 2
    wid = lax.axis_index("s") * nc + lax.axis_index("c")
    pair0 = wid * NPAIR

    pltpu.sync_copy(u_hbm.at[pl.ds(pair0 * K, NPAIR * K)], u_v)

    bufs = ((cum_v0, bp_v0, sem0, out_v0, osem0),
            (cum_v1, bp_v1, sem1, out_v1, osem1))

    _issue2(cum_hbm, bp_hbm, cum_v0, bp_v0, sem0, pair0)
    _issue2(cum_hbm, bp_hbm, cum_v1, bp_v1, sem1, pair0 + 1)

    def compute(j, cum_v, bp_v, out_v):
        jvec = jnp.zeros((16,), jnp.int32) + j
        for k in range(K):
            uk = plsc.load_gather(u_v, [K * jvec + k])
            for q in range(7):
                ri = lax.iota(jnp.int32, 16) + q * 16
                msk = ri < R
                rcl = jnp.minimum(ri, R - 1)
                rc = rcl.astype(jnp.float32) / float(R) + uk / float(R)
                pos = jnp.zeros((16,), jnp.int32)
                for step in (512, 256, 128, 64, 32, 16, 8, 4, 2, 1):
                    val = plsc.load_gather(cum_v, [pos + (step - 1)])
                    pos = pos + jnp.where(val < rc, step, 0)
                bpx = plsc.load_gather(bp_v, [2 * pos])
                bpy = plsc.load_gather(bp_v, [2 * pos + 1])
                oi = k * (R * S) + 2 * rcl
                plsc.store_scatter(out_v, [oi], bpx, mask=msk)
                plsc.store_scatter(out_v, [oi + 1], bpy, mask=msk)

    def pair_step(jj, _):
        for b in range(2):
            cum_v, bp_v, sem, out_v, osem = bufs[b]
            j = 2 * jj + b
            p = pair0 + j
            _drain2(cum_hbm, bp_hbm, cum_v, bp_v, sem)

            @pl.when(jj != 0)
            def _():
                pltpu.make_async_copy(
                    out_v, out_hbm.at[pl.ds(0, OPB)], osem).wait()

            compute(j, cum_v, bp_v, out_v)
            pltpu.async_copy(out_v, out_hbm.at[pl.ds(p * OPB, OPB)], osem)
            pnext = jnp.minimum(p + 2, pair0 + NPAIR - 1)
            _issue2(cum_hbm, bp_hbm, cum_v, bp_v, sem, pnext)
        return ()

    lax.fori_loop(0, NPAIR // 2, pair_step, ())

    for b in range(2):
        cum_v, bp_v, sem, out_v, osem = bufs[b]
        _drain2(cum_hbm, bp_hbm, cum_v, bp_v, sem)
        pltpu.make_async_copy(out_v, out_hbm.at[pl.ds(0, OPB)], osem).wait()


@functools.cache
def _sc_call():
    return pl.kernel(
        _sc_body,
        out_type=jax.ShapeDtypeStruct((N_NODES * B * OPB,), jnp.float32),
        mesh=plsc.VectorSubcoreMesh(core_axis_name="c", subcore_axis_name="s"),
        compiler_params=pltpu.CompilerParams(needs_layout_passes=False),
        scratch_types=[
            pltpu.VMEM((KP,), jnp.float32),
            pltpu.VMEM((KP * S,), jnp.float32),
            pltpu.VMEM((KP,), jnp.float32),
            pltpu.VMEM((KP * S,), jnp.float32),
            pltpu.VMEM((OPB,), jnp.float32),
            pltpu.VMEM((OPB,), jnp.float32),
            pltpu.VMEM((NPAIR * K,), jnp.float32),
            pltpu.SemaphoreType.DMA,
            pltpu.SemaphoreType.DMA,
            pltpu.SemaphoreType.DMA,
            pltpu.SemaphoreType.DMA,
        ],
    )


def kernel(glbl_feats, belief_particles, belief_weights, message_particles,
           u, noise, tw1, tb1, tw2, tb2):
    bw3 = belief_weights.reshape(N_NODES, B, KP)
    bp3 = belief_particles.reshape(N_NODES, B, KP * S)
    noise3 = noise.reshape(N_NODES, Q, NOISE_DIM)
    delta3, cum3 = _tc_call(noise3, bw3, tw1,
                            tb1.reshape(N_NODES, 1, H), tw2,
                            tb2.reshape(N_NODES, 1, S))
    gath = _sc_call()(cum3.reshape(-1), bp3.reshape(-1), u.reshape(-1))
    out = _final_call(message_particles.reshape(N_NODES, B * K, P * S),
                      gath.reshape(N_NODES, B * K, R * S),
                      delta3.reshape(N_NODES, B * K, R * S))
    return out.reshape(N_NODES, B, K, P, S)


# speedup vs baseline: 2.7588x; 2.7588x over previous
import functools

import jax
import jax.numpy as jnp
from jax import lax
from jax.experimental import pallas as pl
from jax.experimental.pallas import tpu as pltpu
from jax.experimental.pallas import tpu_sc as plsc

N_NODES = 8
B = 64
K = 2
P = 512
S = 2
R = 102
NOISE_DIM = 16
H = 64
KP = K * P
Q = B * K * R
OPB = K * R * S
NPAIR = 16


def _tc_body(noise_ref, bw_ref, tw1_ref, tb1_ref, tw2_ref, tb2_ref,
             delta_ref, cum_ref):
    nz = noise_ref[0]
    w1 = tw1_ref[0]
    h = jnp.dot(nz, w1, preferred_element_type=jnp.float32) + tb1_ref[0]
    h = jnp.maximum(h, 0.0)
    d = jnp.dot(h, tw2_ref[0], preferred_element_type=jnp.float32) + tb2_ref[0]
    delta_ref[0] = d

    w = bw_ref[0]
    t = jnp.sum(w, axis=1, keepdims=True)
    c = w / t
    lane = lax.broadcasted_iota(jnp.int32, (B, KP), 1)
    s = 1
    while s < KP:
        c = c + jnp.where(lane >= s, pltpu.roll(c, s, 1), 0.0)
        s *= 2
    cum_ref[0] = c


def _tc_call(noise3, bw3, tw1, tb1r, tw2, tb2r):
    return pl.pallas_call(
        _tc_body,
        grid=(N_NODES,),
        in_specs=[
            pl.BlockSpec((1, Q, NOISE_DIM), lambda i: (i, 0, 0)),
            pl.BlockSpec((1, B, KP), lambda i: (i, 0, 0)),
            pl.BlockSpec((1, NOISE_DIM, H), lambda i: (i, 0, 0)),
            pl.BlockSpec((1, 1, H), lambda i: (i, 0, 0)),
            pl.BlockSpec((1, H, S), lambda i: (i, 0, 0)),
            pl.BlockSpec((1, 1, S), lambda i: (i, 0, 0)),
        ],
        out_specs=[
            pl.BlockSpec((1, Q, S), lambda i: (i, 0, 0)),
            pl.BlockSpec((1, B, KP), lambda i: (i, 0, 0)),
        ],
        out_shape=[
            jax.ShapeDtypeStruct((N_NODES, Q, S), jnp.float32),
            jax.ShapeDtypeStruct((N_NODES, B, KP), jnp.float32),
        ],
    )(noise3, bw3, tw1, tb1r, tw2, tb2r)


def _addclip_body(g_ref, d_ref, o_ref):
    o_ref[...] = jnp.clip(g_ref[...] + d_ref[...], -1.0, 1.0)


def _addclip_call(g2, d2):
    return pl.pallas_call(
        _addclip_body,
        out_shape=jax.ShapeDtypeStruct((N_NODES * B * K, R * S), jnp.float32),
    )(g2, d2)


def _issue2(cum_hbm, bp_hbm, cum_v, bp_v, sem, p):
    pltpu.async_copy(cum_hbm.at[pl.ds(p * KP, KP)], cum_v, sem)
    pltpu.async_copy(bp_hbm.at[p], bp_v, sem)


def _drain2(cum_hbm, bp_hbm, cum_v, bp_v, sem):
    pltpu.make_async_copy(cum_hbm.at[pl.ds(0, KP)], cum_v, sem).wait()
    pltpu.make_async_copy(bp_hbm.at[0], bp_v, sem).wait()


def _sc_body(cum_hbm, bp_hbm, u_hbm, out_hbm,
             cum_v0, bp_v0, cum_v1, bp_v1,
             out_v0, out_v1, u_v,
             sem0, sem1, osem0, osem1):
    nc = 2
    wid = lax.axis_index("s") * nc + lax.axis_index("c")
    pair0 = wid * NPAIR

    pltpu.sync_copy(u_hbm.at[pl.ds(pair0 * K, NPAIR * K)], u_v)

    bufs = ((cum_v0, bp_v0, sem0, out_v0, osem0),
            (cum_v1, bp_v1, sem1, out_v1, osem1))

    _issue2(cum_hbm, bp_hbm, cum_v0, bp_v0, sem0, pair0)
    _issue2(cum_hbm, bp_hbm, cum_v1, bp_v1, sem1, pair0 + 1)

    def compute(j, cum_v, bp_v, out_v):
        jvec = jnp.zeros((16,), jnp.int32) + j
        for k in range(K):
            uk = plsc.load_gather(u_v, [K * jvec + k])
            for q in range(7):
                ri = lax.iota(jnp.int32, 16) + q * 16
                msk = ri < R
                rcl = jnp.minimum(ri, R - 1)
                rc = rcl.astype(jnp.float32) / float(R) + uk / float(R)
                pos = jnp.zeros((16,), jnp.int32)
                for step in (512, 256, 128, 64, 32, 16, 8, 4, 2, 1):
                    val = plsc.load_gather(cum_v, [pos + (step - 1)])
                    pos = pos + jnp.where(val < rc, step, 0)
                bpx = plsc.load_gather(bp_v, [2 * pos])
                bpy = plsc.load_gather(bp_v, [2 * pos + 1])
                oi = k * (R * S) + 2 * rcl
                plsc.store_scatter(out_v, [oi], bpx, mask=msk)
                plsc.store_scatter(out_v, [oi + 1], bpy, mask=msk)

    def pair_step(jj, _):
        for b in range(2):
            cum_v, bp_v, sem, out_v, osem = bufs[b]
            j = 2 * jj + b
            p = pair0 + j
            _drain2(cum_hbm, bp_hbm, cum_v, bp_v, sem)

            @pl.when(jj != 0)
            def _():
                pltpu.make_async_copy(
                    out_v, out_hbm.at[pl.ds(0, OPB)], osem).wait()

            compute(j, cum_v, bp_v, out_v)
            pltpu.async_copy(out_v, out_hbm.at[pl.ds(p * OPB, OPB)], osem)
            pnext = jnp.minimum(p + 2, pair0 + NPAIR - 1)
            _issue2(cum_hbm, bp_hbm, cum_v, bp_v, sem, pnext)
        return ()

    lax.fori_loop(0, NPAIR // 2, pair_step, ())

    for b in range(2):
        cum_v, bp_v, sem, out_v, osem = bufs[b]
        _drain2(cum_hbm, bp_hbm, cum_v, bp_v, sem)
        pltpu.make_async_copy(out_v, out_hbm.at[pl.ds(0, OPB)], osem).wait()


@functools.cache
def _sc_call():
    return pl.kernel(
        _sc_body,
        out_type=jax.ShapeDtypeStruct((N_NODES * B * OPB,), jnp.float32),
        mesh=plsc.VectorSubcoreMesh(core_axis_name="c", subcore_axis_name="s"),
        compiler_params=pltpu.CompilerParams(needs_layout_passes=False),
        scratch_types=[
            pltpu.VMEM((KP,), jnp.float32),
            pltpu.VMEM((KP * S,), jnp.float32),
            pltpu.VMEM((KP,), jnp.float32),
            pltpu.VMEM((KP * S,), jnp.float32),
            pltpu.VMEM((OPB,), jnp.float32),
            pltpu.VMEM((OPB,), jnp.float32),
            pltpu.VMEM((NPAIR * K,), jnp.float32),
            pltpu.SemaphoreType.DMA,
            pltpu.SemaphoreType.DMA,
            pltpu.SemaphoreType.DMA,
            pltpu.SemaphoreType.DMA,
        ],
    )


def kernel(glbl_feats, belief_particles, belief_weights, message_particles,
           u, noise, tw1, tb1, tw2, tb2):
    bw3 = belief_weights.reshape(N_NODES, B, KP)
    bp2 = belief_particles.reshape(N_NODES * B, KP * S)
    noise3 = noise.reshape(N_NODES, Q, NOISE_DIM)
    delta3, cum3 = _tc_call(noise3, bw3, tw1,
                            tb1.reshape(N_NODES, 1, H), tw2,
                            tb2.reshape(N_NODES, 1, S))
    gath = _sc_call()(cum3.reshape(-1), bp2, u.reshape(-1))
    pref = _addclip_call(gath.reshape(N_NODES * B * K, R * S),
                         delta3.reshape(N_NODES * B * K, R * S))
    pref = pref.reshape(N_NODES, B, K, R, S)
    return jnp.concatenate([pref, message_particles[:, :, :, R:, :]], axis=3)


# speedup vs baseline: 3.3041x; 1.1977x over previous
import functools

import jax
import jax.numpy as jnp
from jax import lax
from jax.experimental import pallas as pl
from jax.experimental.pallas import tpu as pltpu
from jax.experimental.pallas import tpu_sc as plsc

N_NODES = 8
B = 64
K = 2
P = 512
S = 2
R = 102
NOISE_DIM = 16
H = 64
KP = K * P
Q = B * K * R
OPB = K * R * S
NPAIR = 16


def _tc_body(noise_ref, bw_ref, tw1_ref, tb1_ref, tw2_ref, tb2_ref,
             delta_ref, cum_ref):
    nz = noise_ref[0]
    w1t = tw1_ref[0].T
    h = jnp.dot(w1t, nz, preferred_element_type=jnp.float32) + tb1_ref[0]
    h = jnp.maximum(h, 0.0)
    w2t = tw2_ref[0].T
    d = jnp.dot(w2t, h, preferred_element_type=jnp.float32) + tb2_ref[0]
    delta_ref[0] = d

    w = bw_ref[0]
    t = jnp.sum(w, axis=1, keepdims=True)
    c = w / t
    lane = lax.broadcasted_iota(jnp.int32, (B, KP), 1)
    s = 1
    while s < KP:
        c = c + jnp.where(lane >= s, pltpu.roll(c, s, 1), 0.0)
        s *= 2
    cum_ref[0] = c


def _tc_call(noise3, bw3, tw1, tb1r, tw2, tb2r):
    return pl.pallas_call(
        _tc_body,
        grid=(N_NODES,),
        in_specs=[
            pl.BlockSpec((1, NOISE_DIM, Q), lambda i: (i, 0, 0)),
            pl.BlockSpec((1, B, KP), lambda i: (i, 0, 0)),
            pl.BlockSpec((1, NOISE_DIM, H), lambda i: (i, 0, 0)),
            pl.BlockSpec((1, H, 1), lambda i: (i, 0, 0)),
            pl.BlockSpec((1, H, S), lambda i: (i, 0, 0)),
            pl.BlockSpec((1, S, 1), lambda i: (i, 0, 0)),
        ],
        out_specs=[
            pl.BlockSpec((1, S, Q), lambda i: (i, 0, 0)),
            pl.BlockSpec((1, B, KP), lambda i: (i, 0, 0)),
        ],
        out_shape=[
            jax.ShapeDtypeStruct((N_NODES, S, Q), jnp.float32),
            jax.ShapeDtypeStruct((N_NODES, B, KP), jnp.float32),
        ],
    )(noise3, bw3, tw1, tb1r, tw2, tb2r)


KR = K * R
DCH = 208


def _issue2(cum_hbm, bp_hbm, d_hbm, cum_v, bp_v, d_v, sem, p):
    pltpu.async_copy(cum_hbm.at[pl.ds(p * KP, KP)], cum_v, sem)
    pltpu.async_copy(bp_hbm.at[p], bp_v, sem)
    n = p >> 6
    b = p & (B - 1)
    base = 2 * n * Q + b * KR
    off = base & 7
    base_al = pl.multiple_of(base - off, 8)
    pltpu.async_copy(d_hbm.at[pl.ds(base_al, DCH)],
                     d_v.at[pl.ds(0, DCH)], sem)
    pltpu.async_copy(d_hbm.at[pl.ds(base_al + Q, DCH)],
                     d_v.at[pl.ds(DCH, DCH)], sem)
    return off


def _drain2(cum_hbm, bp_hbm, d_hbm, cum_v, bp_v, d_v, sem):
    pltpu.make_async_copy(cum_hbm.at[pl.ds(0, KP)], cum_v, sem).wait()
    pltpu.make_async_copy(bp_hbm.at[0], bp_v, sem).wait()
    pltpu.make_async_copy(d_hbm.at[pl.ds(0, DCH)],
                          d_v.at[pl.ds(0, DCH)], sem).wait()
    pltpu.make_async_copy(d_hbm.at[pl.ds(0, DCH)],
                          d_v.at[pl.ds(DCH, DCH)], sem).wait()


def _sc_body(cum_hbm, bp_hbm, d_hbm, u_hbm, out_hbm,
             cum_v0, bp_v0, d_v0, cum_v1, bp_v1, d_v1,
             out_v0, out_v1, u_v,
             sem0, sem1, osem0, osem1):
    nc = 2
    wid = lax.axis_index("s") * nc + lax.axis_index("c")
    pair0 = wid * NPAIR

    pltpu.sync_copy(u_hbm.at[pl.ds(pair0 * K, NPAIR * K)], u_v)

    bufs = ((cum_v0, bp_v0, d_v0, sem0, out_v0, osem0),
            (cum_v1, bp_v1, d_v1, sem1, out_v1, osem1))

    _issue2(cum_hbm, bp_hbm, d_hbm, cum_v0, bp_v0, d_v0, sem0, pair0)
    _issue2(cum_hbm, bp_hbm, d_hbm, cum_v1, bp_v1, d_v1, sem1, pair0 + 1)

    def compute(j, cum_v, bp_v, d_v, out_v, off):
        jvec = jnp.zeros((16,), jnp.int32) + j
        for k in range(K):
            uk = plsc.load_gather(u_v, [K * jvec + k])
            for q in range(7):
                ri = lax.iota(jnp.int32, 16) + q * 16
                msk = ri < R
                rcl = jnp.minimum(ri, R - 1)
                rc = rcl.astype(jnp.float32) / float(R) + uk / float(R)
                pos = jnp.zeros((16,), jnp.int32)
                for step in (512, 256, 128, 64, 32, 16, 8, 4, 2, 1):
                    val = plsc.load_gather(cum_v, [pos + (step - 1)])
                    pos = pos + jnp.where(val < rc, step, 0)
                di = k * R + rcl
                bpx = plsc.load_gather(bp_v, [2 * pos])
                bpy = plsc.load_gather(bp_v, [2 * pos + 1])
                dx = plsc.load_gather(d_v, [off + di])
                dy = plsc.load_gather(d_v, [DCH + off + di])
                vx = jnp.minimum(jnp.maximum(bpx + dx, -1.0), 1.0)
                vy = jnp.minimum(jnp.maximum(bpy + dy, -1.0), 1.0)
                oi = k * (R * S) + 2 * rcl
                plsc.store_scatter(out_v, [oi], vx, mask=msk)
                plsc.store_scatter(out_v, [oi + 1], vy, mask=msk)

    def pair_step(jj, _):
        for b in range(2):
            cum_v, bp_v, d_v, sem, out_v, osem = bufs[b]
            j = 2 * jj + b
            p = pair0 + j
            _drain2(cum_hbm, bp_hbm, d_hbm, cum_v, bp_v, d_v, sem)

            @pl.when(jj != 0)
            def _():
                pltpu.make_async_copy(
                    out_v, out_hbm.at[pl.ds(0, OPB)], osem).wait()

            compute(j, cum_v, bp_v, d_v, out_v, 4 * b)
            pltpu.async_copy(out_v, out_hbm.at[pl.ds(p * OPB, OPB)], osem)
            pnext = jnp.minimum(p + 2, pair0 + NPAIR - 1)
            _issue2(cum_hbm, bp_hbm, d_hbm, cum_v, bp_v, d_v, sem, pnext)
        return ()

    lax.fori_loop(0, NPAIR // 2, pair_step, ())

    for b in range(2):
        cum_v, bp_v, d_v, sem, out_v, osem = bufs[b]
        _drain2(cum_hbm, bp_hbm, d_hbm, cum_v, bp_v, d_v, sem)
        pltpu.make_async_copy(out_v, out_hbm.at[pl.ds(0, OPB)], osem).wait()


@functools.cache
def _sc_call():
    return pl.kernel(
        _sc_body,
        out_type=jax.ShapeDtypeStruct((N_NODES * B * OPB,), jnp.float32),
        mesh=plsc.VectorSubcoreMesh(core_axis_name="c", subcore_axis_name="s"),
        compiler_params=pltpu.CompilerParams(needs_layout_passes=False),
        scratch_types=[
            pltpu.VMEM((KP,), jnp.float32),
            pltpu.VMEM((KP * S,), jnp.float32),
            pltpu.VMEM((S * DCH,), jnp.float32),
            pltpu.VMEM((KP,), jnp.float32),
            pltpu.VMEM((KP * S,), jnp.float32),
            pltpu.VMEM((S * DCH,), jnp.float32),
            pltpu.VMEM((OPB,), jnp.float32),
            pltpu.VMEM((OPB,), jnp.float32),
            pltpu.VMEM((NPAIR * K,), jnp.float32),
            pltpu.SemaphoreType.DMA,
            pltpu.SemaphoreType.DMA,
            pltpu.SemaphoreType.DMA,
            pltpu.SemaphoreType.DMA,
        ],
    )


def kernel(glbl_feats, belief_particles, belief_weights, message_particles,
           u, noise, tw1, tb1, tw2, tb2):
    bw3 = belief_weights.reshape(N_NODES, B, KP)
    bp2 = belief_particles.reshape(N_NODES * B, KP * S)
    noise3 = noise.transpose(0, 4, 1, 2, 3).reshape(N_NODES, NOISE_DIM, Q)
    delta3, cum3 = _tc_call(noise3, bw3, tw1,
                            tb1.reshape(N_NODES, H, 1), tw2,
                            tb2.reshape(N_NODES, S, 1))
    pref = _sc_call()(cum3.reshape(-1), bp2, delta3.reshape(-1),
                      u.reshape(-1))
    pref = pref.reshape(N_NODES, B, K, R, S)
    return jnp.concatenate([pref, message_particles[:, :, :, R:, :]], axis=3)
